# baseline (speedup 1.0000x reference)
import jax
import jax.numpy as jnp
from jax.experimental import pallas as pl

_N = 4096
_D = 256
_K = 33
_R = 256


def _topk_body(xrow_ref, xall_ref, rows_ref, cols_ref, vals_ref):
    xr = xrow_ref[...]
    xa = xall_ref[...]

    na = jnp.sqrt(jnp.sum(xa * xa, axis=1, keepdims=True))
    ba = xa / jnp.maximum(na, 1e-6)
    nr = jnp.sqrt(jnp.sum(xr * xr, axis=1, keepdims=True))
    br = xr / jnp.maximum(nr, 1e-6)

    sim = jax.lax.dot_general(
        br, ba, (((1,), (1,)), ((), ())), preferred_element_type=jnp.float32
    )

    col_iota = jax.lax.broadcasted_iota(jnp.int32, (_R, _N), 1)
    neg = jnp.float32(-jnp.inf)
    idxs = []
    vals = []
    vp = None
    for t in range(_K):
        if t == 0:
            cur = sim
        else:
            vpc = vp[:, None]
            elig = (sim < vpc) | ((sim == vpc) & (col_iota > jp[:, None]))
            cur = jnp.where(elig, sim, neg)
        m = jnp.max(cur, axis=1)
        idx = jnp.min(jnp.where(cur == m[:, None], col_iota, _N), axis=1)
        idxs.append(idx)
        vals.append(m)
        vp, jp = m, idx

    idx = jnp.stack(idxs, axis=1)
    val = jnp.stack(vals, axis=1)

    rank = jnp.zeros((_R, _K), jnp.int32)
    for t in range(_K):
        rank = rank + (idx[:, t : t + 1] < idx).astype(jnp.int32)
    lane = jax.lax.broadcasted_iota(jnp.int32, (_R, _K), 1)
    scol = jnp.zeros((_R, _K), jnp.int32)
    sval = jnp.zeros((_R, _K), jnp.float32)
    for t in range(_K):
        onehot = rank[:, t : t + 1] == lane
        scol = jnp.where(onehot, idx[:, t : t + 1], scol)
        sval = jnp.where(onehot, val[:, t : t + 1], sval)

    i = pl.program_id(0)
    rows_ref[...] = jax.lax.broadcasted_iota(jnp.int32, (_R, _K), 0) + i * _R
    cols_ref[...] = scol
    vals_ref[...] = sval


def kernel(memory_value):
    grid = _N // _R
    rows, cols, vals = pl.pallas_call(
        _topk_body,
        grid=(grid,),
        in_specs=[
            pl.BlockSpec((_R, _D), lambda i: (i, 0)),
            pl.BlockSpec((_N, _D), lambda i: (0, 0)),
        ],
        out_specs=[
            pl.BlockSpec((_R, _K), lambda i: (i, 0)),
            pl.BlockSpec((_R, _K), lambda i: (i, 0)),
            pl.BlockSpec((_R, _K), lambda i: (i, 0)),
        ],
        out_shape=[
            jax.ShapeDtypeStruct((_N, _K), jnp.int32),
            jax.ShapeDtypeStruct((_N, _K), jnp.int32),
            jax.ShapeDtypeStruct((_N, _K), jnp.float32),
        ],
    )(memory_value, memory_value)

    edge_index = jnp.stack([rows.reshape(-1), cols.reshape(-1)]).astype(jnp.int64)
    edge_weight = vals.reshape(-1)
    return (edge_index, edge_weight)


# speedup vs baseline: 1.2117x; 1.2117x over previous
import jax
import jax.numpy as jnp
from jax.experimental import pallas as pl

_N = 4096
_D = 256
_K = 33
_R = 256


def _topk_body(xrow_ref, xall_ref, rows_ref, cols_ref, vals_ref):
    xr = xrow_ref[...]
    xa = xall_ref[...]

    na = jnp.sqrt(jnp.sum(xa * xa, axis=1, keepdims=True))
    ba = xa / jnp.maximum(na, 1e-6)
    nr = jnp.sqrt(jnp.sum(xr * xr, axis=1, keepdims=True))
    br = xr / jnp.maximum(nr, 1e-6)

    sim = jax.lax.dot_general(
        br, ba, (((1,), (1,)), ((), ())), preferred_element_type=jnp.float32
    )

    col_iota = jax.lax.broadcasted_iota(jnp.int32, (_R, _N), 1)
    idxs = []
    vals = []
    s = sim
    for _ in range(_K):
        idx = jnp.argmax(s, axis=1).astype(jnp.int32)
        m = jnp.max(s, axis=1)
        idxs.append(idx)
        vals.append(m)
        s = jnp.where(col_iota == idx[:, None], -jnp.inf, s)

    idx = jnp.stack(idxs, axis=1)
    val = jnp.stack(vals, axis=1)

    rank = jnp.zeros((_R, _K), jnp.int32)
    for t in range(_K):
        rank = rank + (idx[:, t : t + 1] < idx).astype(jnp.int32)
    lane = jax.lax.broadcasted_iota(jnp.int32, (_R, _K), 1)
    scol = jnp.zeros((_R, _K), jnp.int32)
    sval = jnp.zeros((_R, _K), jnp.float32)
    for t in range(_K):
        onehot = rank[:, t : t + 1] == lane
        scol = jnp.where(onehot, idx[:, t : t + 1], scol)
        sval = jnp.where(onehot, val[:, t : t + 1], sval)

    i = pl.program_id(0)
    rows_ref[...] = jax.lax.broadcasted_iota(jnp.int32, (_R, _K), 0) + i * _R
    cols_ref[...] = scol
    vals_ref[...] = sval


def kernel(memory_value):
    grid = _N // _R
    rows, cols, vals = pl.pallas_call(
        _topk_body,
        grid=(grid,),
        in_specs=[
            pl.BlockSpec((_R, _D), lambda i: (i, 0)),
            pl.BlockSpec((_N, _D), lambda i: (0, 0)),
        ],
        out_specs=[
            pl.BlockSpec((_R, _K), lambda i: (i, 0)),
            pl.BlockSpec((_R, _K), lambda i: (i, 0)),
            pl.BlockSpec((_R, _K), lambda i: (i, 0)),
        ],
        out_shape=[
            jax.ShapeDtypeStruct((_N, _K), jnp.int32),
            jax.ShapeDtypeStruct((_N, _K), jnp.int32),
            jax.ShapeDtypeStruct((_N, _K), jnp.float32),
        ],
    )(memory_value, memory_value)

    edge_index = jnp.stack([rows.reshape(-1), cols.reshape(-1)]).astype(jnp.int64)
    edge_weight = vals.reshape(-1)
    return (edge_index, edge_weight)


# speedup vs baseline: 1.6030x; 1.3230x over previous
import jax
import jax.numpy as jnp
from jax.experimental import pallas as pl

_N = 4096
_D = 256
_K = 33
_R = 512


def _topk_body(xrow_ref, xall_ref, rows_ref, cols_ref, vals_ref):
    xr = xrow_ref[...]
    xa = xall_ref[...]

    na = jnp.sqrt(jnp.sum(xa * xa, axis=1, keepdims=True))
    ba = xa / jnp.maximum(na, 1e-6)
    nr = jnp.sqrt(jnp.sum(xr * xr, axis=1, keepdims=True))
    br = xr / jnp.maximum(nr, 1e-6)

    sim = jax.lax.dot_general(
        br, ba, (((1,), (1,)), ((), ())), preferred_element_type=jnp.float32
    )

    col_iota = jax.lax.broadcasted_iota(jnp.int32, (_R, _N), 1)
    idxs = []
    vals = []
    s = sim
    for _ in range(_K):
        m = jnp.max(s, axis=1)
        hit = s == m[:, None]
        idx = jnp.min(jnp.where(hit, col_iota, _N), axis=1)
        idxs.append(idx)
        vals.append(m)
        s = jnp.where(col_iota == idx[:, None], -jnp.inf, s)

    idx = jnp.stack(idxs, axis=1)
    val = jnp.stack(vals, axis=1)

    rank = jnp.zeros((_R, _K), jnp.int32)
    for t in range(_K):
        rank = rank + (idx[:, t : t + 1] < idx).astype(jnp.int32)
    lane = jax.lax.broadcasted_iota(jnp.int32, (_R, _K), 1)
    scol = jnp.zeros((_R, _K), jnp.int32)
    sval = jnp.zeros((_R, _K), jnp.float32)
    for t in range(_K):
        onehot = rank[:, t : t + 1] == lane
        scol = jnp.where(onehot, idx[:, t : t + 1], scol)
        sval = jnp.where(onehot, val[:, t : t + 1], sval)

    i = pl.program_id(0)
    rows_ref[...] = jax.lax.broadcasted_iota(jnp.int32, (_R, _K), 0) + i * _R
    cols_ref[...] = scol
    vals_ref[...] = sval


def kernel(memory_value):
    grid = _N // _R
    rows, cols, vals = pl.pallas_call(
        _topk_body,
        grid=(grid,),
        in_specs=[
            pl.BlockSpec((_R, _D), lambda i: (i, 0)),
            pl.BlockSpec((_N, _D), lambda i: (0, 0)),
        ],
        out_specs=[
            pl.BlockSpec((_R, _K), lambda i: (i, 0)),
            pl.BlockSpec((_R, _K), lambda i: (i, 0)),
            pl.BlockSpec((_R, _K), lambda i: (i, 0)),
        ],
        out_shape=[
            jax.ShapeDtypeStruct((_N, _K), jnp.int32),
            jax.ShapeDtypeStruct((_N, _K), jnp.int32),
            jax.ShapeDtypeStruct((_N, _K), jnp.float32),
        ],
    )(memory_value, memory_value)

    edge_index = jnp.stack([rows.reshape(-1), cols.reshape(-1)]).astype(jnp.int64)
    edge_weight = vals.reshape(-1)
    return (edge_index, edge_weight)


# speedup vs baseline: 1.6956x; 1.0577x over previous
import jax
import jax.numpy as jnp
from jax.experimental import pallas as pl

_N = 4096
_D = 256
_K = 33
_R = 1024


def _topk_body(xrow_ref, xall_ref, rows_ref, cols_ref, vals_ref):
    xr = xrow_ref[...]
    xa = xall_ref[...]

    na = jnp.sqrt(jnp.sum(xa * xa, axis=1, keepdims=True))
    ba = xa / jnp.maximum(na, 1e-6)
    nr = jnp.sqrt(jnp.sum(xr * xr, axis=1, keepdims=True))
    br = xr / jnp.maximum(nr, 1e-6)

    sim = jax.lax.dot_general(
        br, ba, (((1,), (1,)), ((), ())), preferred_element_type=jnp.float32
    )

    col_iota = jax.lax.broadcasted_iota(jnp.int32, (_R, _N), 1)
    idxs = []
    vals = []
    s = sim
    for _ in range(_K):
        m = jnp.max(s, axis=1)
        hit = s == m[:, None]
        idx = jnp.min(jnp.where(hit, col_iota, _N), axis=1)
        idxs.append(idx)
        vals.append(m)
        s = jnp.where(col_iota == idx[:, None], -jnp.inf, s)

    idx = jnp.stack(idxs, axis=1)
    val = jnp.stack(vals, axis=1)

    rank = jnp.zeros((_R, _K), jnp.int32)
    for t in range(_K):
        rank = rank + (idx[:, t : t + 1] < idx).astype(jnp.int32)
    lane = jax.lax.broadcasted_iota(jnp.int32, (_R, _K), 1)
    scol = jnp.zeros((_R, _K), jnp.int32)
    sval = jnp.zeros((_R, _K), jnp.float32)
    for t in range(_K):
        onehot = rank[:, t : t + 1] == lane
        scol = jnp.where(onehot, idx[:, t : t + 1], scol)
        sval = jnp.where(onehot, val[:, t : t + 1], sval)

    i = pl.program_id(0)
    rows_ref[...] = jax.lax.broadcasted_iota(jnp.int32, (_R, _K), 0) + i * _R
    cols_ref[...] = scol
    vals_ref[...] = sval


def kernel(memory_value):
    grid = _N // _R
    rows, cols, vals = pl.pallas_call(
        _topk_body,
        grid=(grid,),
        in_specs=[
            pl.BlockSpec((_R, _D), lambda i: (i, 0)),
            pl.BlockSpec((_N, _D), lambda i: (0, 0)),
        ],
        out_specs=[
            pl.BlockSpec((_R, _K), lambda i: (i, 0)),
            pl.BlockSpec((_R, _K), lambda i: (i, 0)),
            pl.BlockSpec((_R, _K), lambda i: (i, 0)),
        ],
        out_shape=[
            jax.ShapeDtypeStruct((_N, _K), jnp.int32),
            jax.ShapeDtypeStruct((_N, _K), jnp.int32),
            jax.ShapeDtypeStruct((_N, _K), jnp.float32),
        ],
    )(memory_value, memory_value)

    edge_index = jnp.stack([rows.reshape(-1), cols.reshape(-1)]).astype(jnp.int64)
    edge_weight = vals.reshape(-1)
    return (edge_index, edge_weight)


# speedup vs baseline: 1.7618x; 1.0390x over previous
import functools

import jax
import jax.numpy as jnp
from jax import lax
from jax.experimental import pallas as pl
from jax.experimental.pallas import tpu as pltpu
from jax.experimental.pallas import tpu_sc as plsc

_N = 4096
_D = 256
_K = 33
_R = 1024
_W = 48
_NWORK = 32
_RPW = _N // _NWORK


def _topk_body(xrow_ref, xall_ref, rows_ref, cols_ref, vals_ref):
    xr = xrow_ref[...]
    xa = xall_ref[...]

    na = jnp.sqrt(jnp.sum(xa * xa, axis=1, keepdims=True))
    ba = xa / jnp.maximum(na, 1e-6)
    nr = jnp.sqrt(jnp.sum(xr * xr, axis=1, keepdims=True))
    br = xr / jnp.maximum(nr, 1e-6)

    sim = jax.lax.dot_general(
        br, ba, (((1,), (1,)), ((), ())), preferred_element_type=jnp.float32
    )

    col_iota = jax.lax.broadcasted_iota(jnp.int32, (_R, _N), 1)
    idxs = []
    vals = []
    s = sim
    for _ in range(_K):
        m = jnp.max(s, axis=1)
        hit = s == m[:, None]
        idx = jnp.min(jnp.where(hit, col_iota, _N), axis=1)
        idxs.append(idx)
        vals.append(m)
        s = jnp.where(col_iota == idx[:, None], -jnp.inf, s)

    idx = jnp.stack(idxs, axis=1)
    val = jnp.stack(vals, axis=1)

    lane_pad = jax.lax.broadcasted_iota(jnp.int32, (_R, _W - _K), 1)
    idx48 = jnp.concatenate([idx, lane_pad + jnp.int32(1 << 20)], axis=1)
    val48 = jnp.concatenate([val, jnp.zeros((_R, _W - _K), jnp.float32)], axis=1)

    i = pl.program_id(0)
    rows_ref[...] = jax.lax.broadcasted_iota(jnp.int32, (_R, _K), 0) + i * _R
    cols_ref[...] = idx48
    vals_ref[...] = val48


def _sc_sort_body(idx_hbm, val_hbm, oc_hbm, ov_hbm, idx_v, val_v, oc_v, ov_v):
    wid = lax.axis_index("s") * 2 + lax.axis_index("c")
    base = wid * _RPW * _W
    pltpu.sync_copy(idx_hbm.at[pl.ds(base, _RPW * _W)], idx_v)
    pltpu.sync_copy(val_hbm.at[pl.ds(base, _RPW * _W)], val_v)

    def row(r, carry):
        off = r * _W
        ks = [idx_v[pl.ds(off + 16 * b, 16)] for b in range(3)]
        vs = [val_v[pl.ds(off + 16 * b, 16)] for b in range(3)]
        bcast = [
            jnp.broadcast_to(ks[b][l], (16,)) for b in range(3) for l in range(16)
        ]
        for b in range(3):
            rk = jnp.zeros((16,), jnp.int32)
            for kv in bcast:
                rk = rk + (kv < ks[b]).astype(jnp.int32)
            plsc.store_scatter(oc_v, [rk + off], ks[b])
            plsc.store_scatter(ov_v, [rk + off], vs[b])
        return carry

    lax.fori_loop(0, _RPW, row, 0)
    pltpu.sync_copy(oc_v, oc_hbm.at[pl.ds(base, _RPW * _W)])
    pltpu.sync_copy(ov_v, ov_hbm.at[pl.ds(base, _RPW * _W)])


_sc_sort = functools.partial(
    pl.kernel,
    out_type=[
        jax.ShapeDtypeStruct((_N * _W,), jnp.int32),
        jax.ShapeDtypeStruct((_N * _W,), jnp.float32),
    ],
    mesh=plsc.VectorSubcoreMesh(core_axis_name="c", subcore_axis_name="s"),
    compiler_params=pltpu.CompilerParams(needs_layout_passes=False),
    scratch_types=[
        pltpu.VMEM((_RPW * _W,), jnp.int32),
        pltpu.VMEM((_RPW * _W,), jnp.float32),
        pltpu.VMEM((_RPW * _W,), jnp.int32),
        pltpu.VMEM((_RPW * _W,), jnp.float32),
    ],
)(_sc_sort_body)


def kernel(memory_value):
    grid = _N // _R
    rows, cols, vals = pl.pallas_call(
        _topk_body,
        grid=(grid,),
        in_specs=[
            pl.BlockSpec((_R, _D), lambda i: (i, 0)),
            pl.BlockSpec((_N, _D), lambda i: (0, 0)),
        ],
        out_specs=[
            pl.BlockSpec((_R, _K), lambda i: (i, 0)),
            pl.BlockSpec((_R, _W), lambda i: (i, 0)),
            pl.BlockSpec((_R, _W), lambda i: (i, 0)),
        ],
        out_shape=[
            jax.ShapeDtypeStruct((_N, _K), jnp.int32),
            jax.ShapeDtypeStruct((_N, _W), jnp.int32),
            jax.ShapeDtypeStruct((_N, _W), jnp.float32),
        ],
    )(memory_value, memory_value)

    oc, ov = _sc_sort(cols.reshape(-1), vals.reshape(-1))
    edge_index = jnp.stack(
        [rows.reshape(-1), oc.reshape(_N, _W)[:, :_K].reshape(-1)]
    ).astype(jnp.int64)
    edge_weight = ov.reshape(_N, _W)[:, :_K].reshape(-1)
    return (edge_index, edge_weight)
